# baseline (speedup 1.0000x reference)
import functools

import jax
import jax.numpy as jnp
from jax import lax
from jax.experimental import pallas as pl
from jax.experimental.pallas import tpu as pltpu
from jax.experimental.pallas import tpu_sc as plsc

SEQ = 16384
DIM = 64
NUM_TILES = 32
PER_TILE = SEQ // NUM_TILES
CHUNK = 128
NCHUNK = PER_TILE // CHUNK
L = 16


def _sc_pooled_partials(X, E):
    mesh = plsc.VectorSubcoreMesh(core_axis_name="c", subcore_axis_name="s")

    @functools.partial(
        pl.kernel,
        mesh=mesh,
        out_type=jax.ShapeDtypeStruct((NUM_TILES, DIM), jnp.float32),
        scratch_types=[
            pltpu.VMEM((PER_TILE,), jnp.int32),
            pltpu.VMEM((PER_TILE, DIM), jnp.float32),
            pltpu.VMEM((DIM,), jnp.float32),
            pltpu.SemaphoreType.DMA,
        ],
        compiler_params=pltpu.CompilerParams(use_tc_tiling_on_sc=False),
    )
    def sc_kernel(x_hbm, e_hbm, out_hbm, idx_v, rows_v, acc_v, sem):
        wid = lax.axis_index("s") * 2 + lax.axis_index("c")
        base = wid * PER_TILE
        pltpu.sync_copy(x_hbm.at[pl.ds(base, PER_TILE)], idx_v)
        copies = []
        for j in range(NCHUNK):
            copies.append(pltpu.async_copy(
                e_hbm.at[idx_v.at[pl.ds(j * CHUNK, CHUNK)]],
                rows_v.at[pl.ds(j * CHUNK, CHUNK)],
                sem,
            ))
        for c in copies:
            c.wait()

        zero = jnp.zeros((L,), jnp.float32)

        def body(i, acc):
            a0, a1, a2, a3 = acc
            a0 = a0 + jnp.maximum(rows_v[i, pl.ds(0 * L, L)], 0.0)
            a1 = a1 + jnp.maximum(rows_v[i, pl.ds(1 * L, L)], 0.0)
            a2 = a2 + jnp.maximum(rows_v[i, pl.ds(2 * L, L)], 0.0)
            a3 = a3 + jnp.maximum(rows_v[i, pl.ds(3 * L, L)], 0.0)
            return (a0, a1, a2, a3)

        a0, a1, a2, a3 = lax.fori_loop(
            0, PER_TILE, body, (zero, zero, zero, zero), unroll=4
        )
        acc_v[pl.ds(0 * L, L)] = a0
        acc_v[pl.ds(1 * L, L)] = a1
        acc_v[pl.ds(2 * L, L)] = a2
        acc_v[pl.ds(3 * L, L)] = a3
        pltpu.sync_copy(acc_v, out_hbm.at[wid])

    return sc_kernel(X, E)


def _tc_head(partials, W, b2):
    def tc_kernel(p_ref, w_ref, b_ref, o_ref):
        h = jnp.sum(p_ref[...], axis=0, keepdims=True)
        logits = lax.dot_general(
            h, w_ref[...], (((1,), (1,)), ((), ())),
            preferred_element_type=jnp.float32,
        ) + b_ref[...]
        m = jnp.max(logits, axis=1, keepdims=True)
        e = jnp.exp(logits - m)
        o_ref[...] = e / jnp.sum(e, axis=1, keepdims=True)

    return pl.pallas_call(
        tc_kernel,
        out_shape=jax.ShapeDtypeStruct((1, 2), jnp.float32),
    )(partials, W, b2)


def kernel(X, E, W, b):
    X = X.astype(jnp.int32)
    partials = _sc_pooled_partials(X, E)
    out = _tc_head(partials, W, b.reshape(1, 2))
    return out.reshape(2)


# speedup vs baseline: 3.7336x; 3.7336x over previous
import functools

import jax
import jax.numpy as jnp
from jax import lax
from jax.experimental import pallas as pl
from jax.experimental.pallas import tpu as pltpu
from jax.experimental.pallas import tpu_sc as plsc

SEQ = 16384
DIM = 64
VOCAB = 1000000
NUM_CORES = 2
NUM_SUBCORES = 16
PER_SUBCORE = SEQ // NUM_SUBCORES
CHUNK = 128
NCHUNK = PER_SUBCORE // CHUNK
L = 16

BV = 8192
GRID = 123
HALF = 62 * BV
H_PER_TILE = HALF // NUM_SUBCORES


def _sc_multiplicity(X):
    mesh = plsc.VectorSubcoreMesh(core_axis_name="c", subcore_axis_name="s")

    @functools.partial(
        pl.kernel,
        mesh=mesh,
        out_type=jax.ShapeDtypeStruct((NUM_CORES, HALF), jnp.float32),
        scratch_types=[
            pltpu.VMEM((NCHUNK, CHUNK), jnp.int32),
            pltpu.VMEM((CHUNK,), jnp.float32),
            pltpu.VMEM((H_PER_TILE,), jnp.float32),
            pltpu.VMEM_SHARED((HALF + L,), jnp.float32),
        ],
    )
    def sc_kernel(x_hbm, m_hbm, idx_v, ones_v, zeros_v, m_sh):
        cid = lax.axis_index("c")
        sid = lax.axis_index("s")
        base = sid * PER_SUBCORE
        lo = cid * HALF
        dump = HALF + (sid % L)

        for j in range(NCHUNK):
            pltpu.sync_copy(
                x_hbm.at[pl.ds(base + j * CHUNK, CHUNK)], idx_v.at[j]
            )
        for j in range(NCHUNK):
            for k in range(CHUNK // L):
                v = idx_v[j, pl.ds(k * L, L)] - lo
                ok = (v >= 0) & (v < HALF)
                idx_v[j, pl.ds(k * L, L)] = jnp.where(ok, v, dump)

        for k in range(CHUNK // L):
            ones_v[pl.ds(k * L, L)] = jnp.full((L,), 1.0, jnp.float32)

        def zbody(i, _):
            zeros_v[pl.ds(i * L, L)] = jnp.zeros((L,), jnp.float32)
            return 0

        lax.fori_loop(0, H_PER_TILE // L, zbody, 0, unroll=8)

        tslice = pl.ds(sid * H_PER_TILE, H_PER_TILE)
        pltpu.sync_copy(zeros_v, m_sh.at[tslice])
        plsc.subcore_barrier()

        for j in range(NCHUNK):
            pltpu.sync_copy(ones_v, m_sh.at[idx_v.at[j]], add=True)
        plsc.subcore_barrier()

        pltpu.sync_copy(m_sh.at[tslice], m_hbm.at[cid, tslice])

    return sc_kernel(X)


def _tc_pooled_head(ET, m, W, b2):
    def tc_kernel(e_ref, m_ref, w_ref, b_ref, o_ref, acc_ref):
        i = pl.program_id(0)
        cols = jax.lax.broadcasted_iota(jnp.int32, (1, BV), 1) + i * BV
        e = jnp.where(cols < VOCAB, e_ref[...], 0.0)
        mm = jnp.where(i < 62 * 1, m_ref[0:1, :], m_ref[1:2, :])
        contrib = jnp.sum(jnp.maximum(e, 0.0) * mm, axis=1, keepdims=True)

        @pl.when(i == 0)
        def _():
            acc_ref[...] = contrib

        @pl.when(i > 0)
        def _():
            acc_ref[...] = acc_ref[...] + contrib

        @pl.when(i == GRID - 1)
        def _():
            hidden = acc_ref[...]
            logits = lax.dot_general(
                w_ref[...], hidden, (((1,), (0,)), ((), ())),
                preferred_element_type=jnp.float32,
            ) + b_ref[...]
            mx = jnp.max(logits, axis=0, keepdims=True)
            ex = jnp.exp(logits - mx)
            o_ref[...] = ex / jnp.sum(ex, axis=0, keepdims=True)

    return pl.pallas_call(
        tc_kernel,
        grid=(GRID,),
        in_specs=[
            pl.BlockSpec((DIM, BV), lambda i: (0, i)),
            pl.BlockSpec((NUM_CORES, BV), lambda i: (0, i % 62)),
            pl.BlockSpec((2, DIM), lambda i: (0, 0)),
            pl.BlockSpec((2, 1), lambda i: (0, 0)),
        ],
        out_specs=pl.BlockSpec((2, 1), lambda i: (0, 0)),
        out_shape=jax.ShapeDtypeStruct((2, 1), jnp.float32),
        scratch_shapes=[pltpu.VMEM((DIM, 1), jnp.float32)],
    )(ET, m, W, b2)


def kernel(X, E, W, b):
    X = X.astype(jnp.int32)
    m = _sc_multiplicity(X)
    out = _tc_pooled_head(E.T, m, W, b.reshape(2, 1))
    return out.reshape(2)
